# baseline (speedup 1.0000x reference)
import functools
import jax
import jax.numpy as jnp
from jax import lax
from jax.experimental import pallas as pl
from jax.experimental.pallas import tpu as pltpu
from jax.experimental.pallas import tpu_sc as plsc

_N, _D, _OUT = 16384, 4096, 4
_NC, _NS, _L = 2, 16, 16
_NW = _NC * _NS
_R = 8


def _sc_call(x, wt, comb16):
    S = x.shape[0]
    rw = S // _NW
    nchunks = rw // _R
    mesh = plsc.VectorSubcoreMesh(
        core_axis_name="c", subcore_axis_name="s",
        num_cores=_NC, num_subcores=_NS,
    )

    @functools.partial(
        pl.kernel,
        out_type=jax.ShapeDtypeStruct((S // 4, 4 * _OUT), jnp.float32),
        mesh=mesh,
        scratch_types=[
            pltpu.VMEM((2, _R, _D), jnp.float32),
            pltpu.VMEM((_OUT, _D), jnp.float32),
            pltpu.VMEM((rw // 4, 4 * _OUT), jnp.float32),
            pltpu.VMEM((_L,), jnp.float32),
            pltpu.SemaphoreType.DMA,
            pltpu.SemaphoreType.DMA,
            pltpu.SemaphoreType.DMA,
        ],
    )
    def sck(x_hbm, wt_hbm, comb_hbm, out_hbm, xbuf, wbuf, obuf, cbuf,
            sem0, sem1, osem):
        wid = lax.axis_index("s") * _NC + lax.axis_index("c")
        base = pl.multiple_of(wid * rw, rw)
        pltpu.sync_copy(wt_hbm, wbuf)
        pltpu.sync_copy(comb_hbm, cbuf)
        sems = (sem0, sem1)

        def start_copy(ci, b):
            pltpu.make_async_copy(
                x_hbm.at[pl.ds(pl.multiple_of(base + ci * _R, _R), _R)], xbuf.at[b], sems[b]
            ).start()

        start_copy(0, 0)

        zero = jnp.zeros((_L,), jnp.float32)
        init = tuple(tuple(zero for _ in range(_OUT)) for _ in range(_R))

        def make_kbody(b):
            def kbody(k, accs):
                off = k * _L
                ws = tuple(wbuf[j, pl.ds(off, _L)] for j in range(_OUT))
                new = []
                for r in range(_R):
                    xv = xbuf[b, r, pl.ds(off, _L)]
                    new.append(
                        tuple(accs[r][j] + xv * ws[j] for j in range(_OUT))
                    )
                return tuple(new)
            return kbody

        gd = lax.GatherDimensionNumbers(
            offset_dims=(), collapsed_slice_dims=(0,), start_index_map=(0,))

        def lane_sum(v, lanes):
            for k in (1, 2, 4, 8):
                perm = jnp.bitwise_xor(lanes, k)
                v = v + lax.gather(
                    v, perm[:, None], gd, slice_sizes=(1,),
                    mode=lax.GatherScatterMode.PROMISE_IN_BOUNDS)
            return v

        def pair_body(p, _):
            cv = cbuf[pl.ds(0, _L)]
            lanes = lax.iota(jnp.int32, _L)
            for b in range(2):
                ci = p * 2 + b
                pltpu.make_async_copy(
                    x_hbm.at[pl.ds(pl.multiple_of(base + ci * _R, _R), _R)], xbuf.at[b], sems[b]
                ).wait()

                @pl.when(ci + 1 < nchunks)
                def _():
                    start_copy(ci + 1, 1 - b)

                accs = lax.fori_loop(0, _D // _L, make_kbody(b), init)
                for g in range(_R // 4):
                    vec = jnp.zeros((_L,), jnp.float32)
                    for l in range(_L):
                        r, j = g * 4 + l // 4, l % 4
                        s = lane_sum(accs[r][j], lanes) + cv[j]
                        vec = jnp.where(lanes == l, s, vec)
                    obuf[ci * (_R // 4) + g, :] = vec
            return 0

        lax.fori_loop(0, nchunks // 2, pair_body, 0)
        cp = pltpu.make_async_copy(
            obuf, out_hbm.at[pl.ds(pl.multiple_of(base // 4, rw // 4), rw // 4)], osem)
        cp.start()
        cp.wait()

    return sck(x, wt, comb16).reshape(S, _OUT)


_BN = 512
_SC_ROWS = 4096


def _tc_body(x_ref, w_ref, b_ref, o_ref):
    xh = x_ref[...].astype(jnp.bfloat16)
    o_ref[...] = (
        jnp.dot(xh, w_ref[...], preferred_element_type=jnp.float32)
        + b_ref[...]
    )


def _tc_call(x, w, comb):
    n = x.shape[0]
    return pl.pallas_call(
        _tc_body,
        grid=(n // _BN,),
        in_specs=[
            pl.BlockSpec((_BN, _D), lambda i: (i, 0)),
            pl.BlockSpec((_D, _OUT), lambda i: (0, 0)),
            pl.BlockSpec((1, _OUT), lambda i: (0, 0)),
        ],
        out_specs=pl.BlockSpec((_BN, _OUT), lambda i: (i, 0)),
        out_shape=jax.ShapeDtypeStruct((n, _OUT), jnp.float32),
        compiler_params=pltpu.CompilerParams(
            dimension_semantics=("arbitrary",),
        ),
    )(x, w, comb)


def kernel(inputs, kernel, bias, a, c):
    comb = bias + a + c
    comb16 = jnp.pad(comb, (0, _L - _OUT))
    wt = kernel.T
    n_tc = _N - _SC_ROWS
    sc_out = _sc_call(inputs[n_tc:], wt, comb16)
    tc_out = _tc_call(inputs[:n_tc], kernel.astype(jnp.bfloat16),
                      comb.reshape(1, _OUT))
    return jnp.concatenate([tc_out, sc_out], axis=0)


# speedup vs baseline: 2.4567x; 2.4567x over previous
import functools
import jax
import jax.numpy as jnp
from jax import lax
from jax.experimental import pallas as pl
from jax.experimental.pallas import tpu as pltpu
from jax.experimental.pallas import tpu_sc as plsc

_N, _D, _OUT = 16384, 4096, 4
_NC, _NS, _L = 2, 16, 16
_NW = _NC * _NS
_R = 8


def _sc_call(x, wt, comb16, row0, S):
    rw = S // _NW
    nchunks = rw // _R
    mesh = plsc.VectorSubcoreMesh(
        core_axis_name="c", subcore_axis_name="s",
        num_cores=_NC, num_subcores=_NS,
    )

    @functools.partial(
        pl.kernel,
        out_type=jax.ShapeDtypeStruct((S // 4, 4 * _OUT), jnp.float32),
        mesh=mesh,
        scratch_types=[
            pltpu.VMEM((2, _R, _D), jnp.float32),
            pltpu.VMEM((_OUT, _D), jnp.float32),
            pltpu.VMEM((rw // 4, 4 * _OUT), jnp.float32),
            pltpu.VMEM((_L,), jnp.float32),
            pltpu.SemaphoreType.DMA,
            pltpu.SemaphoreType.DMA,
            pltpu.SemaphoreType.DMA,
        ],
    )
    def sck(x_hbm, wt_hbm, comb_hbm, out_hbm, xbuf, wbuf, obuf, cbuf,
            sem0, sem1, osem):
        wid = lax.axis_index("s") * _NC + lax.axis_index("c")
        base = pl.multiple_of(row0 + wid * rw, _R)
        obase = pl.multiple_of(wid * (rw // 4), rw // 4)
        pltpu.sync_copy(wt_hbm, wbuf)
        pltpu.sync_copy(comb_hbm, cbuf)
        sems = (sem0, sem1)

        def start_copy(ci, b):
            pltpu.make_async_copy(
                x_hbm.at[pl.ds(pl.multiple_of(base + ci * _R, _R), _R)],
                xbuf.at[b], sems[b],
            ).start()

        start_copy(0, 0)

        zero = jnp.zeros((_L,), jnp.float32)
        init = tuple(tuple(zero for _ in range(_OUT)) for _ in range(_R))

        def make_kbody(b):
            def kbody(k, accs):
                off = k * _L
                ws = tuple(wbuf[j, pl.ds(off, _L)] for j in range(_OUT))
                new = []
                for r in range(_R):
                    xv = xbuf[b, r, pl.ds(off, _L)]
                    new.append(
                        tuple(accs[r][j] + xv * ws[j] for j in range(_OUT))
                    )
                return tuple(new)
            return kbody

        gd = lax.GatherDimensionNumbers(
            offset_dims=(), collapsed_slice_dims=(0,), start_index_map=(0,))

        def lane_sum(v, lanes):
            for k in (1, 2, 4, 8):
                perm = jnp.bitwise_xor(lanes, k)
                v = v + lax.gather(
                    v, perm[:, None], gd, slice_sizes=(1,),
                    mode=lax.GatherScatterMode.PROMISE_IN_BOUNDS)
            return v

        def pair_body(p, _):
            cv = cbuf[pl.ds(0, _L)]
            lanes = lax.iota(jnp.int32, _L)
            for b in range(2):
                ci = p * 2 + b
                pltpu.make_async_copy(
                    x_hbm.at[pl.ds(pl.multiple_of(base + ci * _R, _R), _R)], xbuf.at[b], sems[b]
                ).wait()

                @pl.when(ci + 1 < nchunks)
                def _():
                    start_copy(ci + 1, 1 - b)

                accs = lax.fori_loop(0, _D // _L, make_kbody(b), init)
                for g in range(_R // 4):
                    vec = jnp.zeros((_L,), jnp.float32)
                    for l in range(_L):
                        r, j = g * 4 + l // 4, l % 4
                        s = lane_sum(accs[r][j], lanes) + cv[j]
                        vec = jnp.where(lanes == l, s, vec)
                    obuf[ci * (_R // 4) + g, :] = vec
            return 0

        lax.fori_loop(0, nchunks // 2, pair_body, 0)
        cp = pltpu.make_async_copy(
            obuf, out_hbm.at[pl.ds(obase, rw // 4)], osem)
        cp.start()
        cp.wait()

    return sck(x, wt, comb16).reshape(S, _OUT)


_BN = 512
_SC_ROWS = 4096


def _tc_body(x_ref, w_ref, b_ref, o_ref):
    xh = x_ref[...].astype(jnp.bfloat16)
    o_ref[...] = (
        jnp.dot(xh, w_ref[...], preferred_element_type=jnp.float32)
        + b_ref[...]
    )


def _tc_call(x, w, comb, n):
    return pl.pallas_call(
        _tc_body,
        grid=(n // _BN,),
        in_specs=[
            pl.BlockSpec((_BN, _D), lambda i: (i, 0)),
            pl.BlockSpec((_D, _OUT), lambda i: (0, 0)),
            pl.BlockSpec((1, _OUT), lambda i: (0, 0)),
        ],
        out_specs=pl.BlockSpec((_BN, _OUT), lambda i: (i, 0)),
        out_shape=jax.ShapeDtypeStruct((n, _OUT), jnp.float32),
        compiler_params=pltpu.CompilerParams(
            dimension_semantics=("arbitrary",),
        ),
    )(x, w, comb)


def kernel(inputs, kernel, bias, a, c):
    comb = bias + a + c
    comb16 = jnp.pad(comb, (0, _L - _OUT))
    wt = kernel.T
    n_tc = _N - _SC_ROWS
    sc_out = _sc_call(inputs, wt, comb16, n_tc, _SC_ROWS)
    tc_out = _tc_call(inputs, kernel.astype(jnp.bfloat16),
                      comb.reshape(1, _OUT), n_tc)
    return jnp.concatenate([tc_out, sc_out], axis=0)


# speedup vs baseline: 2.5588x; 1.0416x over previous
import functools
import jax
import jax.numpy as jnp
from jax import lax
from jax.experimental import pallas as pl
from jax.experimental.pallas import tpu as pltpu
from jax.experimental.pallas import tpu_sc as plsc

_N, _D, _OUT = 16384, 4096, 4
_NC, _NS, _L = 2, 16, 16
_NW = _NC * _NS
_R = 8


def _sc_call(x, wt, comb16, row0, S):
    rw = S // _NW
    nchunks = rw // _R
    mesh = plsc.VectorSubcoreMesh(
        core_axis_name="c", subcore_axis_name="s",
        num_cores=_NC, num_subcores=_NS,
    )

    @functools.partial(
        pl.kernel,
        out_type=jax.ShapeDtypeStruct((S // 4, 4 * _OUT), jnp.float32),
        mesh=mesh,
        scratch_types=[
            pltpu.VMEM((2, _R, _D), jnp.float32),
            pltpu.VMEM((_OUT, _D), jnp.float32),
            pltpu.VMEM((rw // 4, 4 * _OUT), jnp.float32),
            pltpu.VMEM((_L,), jnp.float32),
            pltpu.SemaphoreType.DMA,
            pltpu.SemaphoreType.DMA,
            pltpu.SemaphoreType.DMA,
        ],
    )
    def sck(x_hbm, wt_hbm, comb_hbm, out_hbm, xbuf, wbuf, obuf, cbuf,
            sem0, sem1, osem):
        wid = lax.axis_index("s") * _NC + lax.axis_index("c")
        base = pl.multiple_of(row0 + wid * rw, _R)
        obase = pl.multiple_of(wid * (rw // 4), rw // 4)
        pltpu.sync_copy(wt_hbm, wbuf)
        pltpu.sync_copy(comb_hbm, cbuf)
        sems = (sem0, sem1)

        def start_copy(ci, b):
            pltpu.make_async_copy(
                x_hbm.at[pl.ds(pl.multiple_of(base + ci * _R, _R), _R)],
                xbuf.at[b], sems[b],
            ).start()

        start_copy(0, 0)

        zero = jnp.zeros((_L,), jnp.float32)
        init = tuple(tuple(zero for _ in range(_OUT)) for _ in range(_R))

        def make_kbody(b):
            def kbody(k, accs):
                off = k * _L
                ws = tuple(wbuf[j, pl.ds(off, _L)] for j in range(_OUT))
                new = []
                for r in range(_R):
                    xv = xbuf[b, r, pl.ds(off, _L)]
                    new.append(
                        tuple(accs[r][j] + xv * ws[j] for j in range(_OUT))
                    )
                return tuple(new)
            return kbody

        gd = lax.GatherDimensionNumbers(
            offset_dims=(), collapsed_slice_dims=(0,), start_index_map=(0,))

        def lane_sum(v, lanes):
            for k in (1, 2, 4, 8):
                perm = jnp.bitwise_xor(lanes, k)
                v = v + lax.gather(
                    v, perm[:, None], gd, slice_sizes=(1,),
                    mode=lax.GatherScatterMode.PROMISE_IN_BOUNDS)
            return v

        def pair_body(p, _):
            cv = cbuf[pl.ds(0, _L)]
            lanes = lax.iota(jnp.int32, _L)
            for b in range(2):
                ci = p * 2 + b
                pltpu.make_async_copy(
                    x_hbm.at[pl.ds(pl.multiple_of(base + ci * _R, _R), _R)], xbuf.at[b], sems[b]
                ).wait()

                @pl.when(ci + 1 < nchunks)
                def _():
                    start_copy(ci + 1, 1 - b)

                accs = lax.fori_loop(0, _D // _L, make_kbody(b), init)
                for g in range(_R // 4):
                    vec = jnp.zeros((_L,), jnp.float32)
                    for l in range(_L):
                        r, j = g * 4 + l // 4, l % 4
                        s = lane_sum(accs[r][j], lanes) + cv[j]
                        vec = jnp.where(lanes == l, s, vec)
                    obuf[ci * (_R // 4) + g, :] = vec
            return 0

        lax.fori_loop(0, nchunks // 2, pair_body, 0)
        cp = pltpu.make_async_copy(
            obuf, out_hbm.at[pl.ds(obase, rw // 4)], osem)
        cp.start()
        cp.wait()

    return sck(x, wt, comb16).reshape(S, _OUT)


_BN = 512
_SC_ROWS = 2048


def _tc_body(x_ref, w_ref, b_ref, o_ref):
    xh = x_ref[...].astype(jnp.bfloat16)
    o_ref[...] = (
        jnp.dot(xh, w_ref[...], preferred_element_type=jnp.float32)
        + b_ref[...]
    )


def _tc_call(x, w, comb, n):
    return pl.pallas_call(
        _tc_body,
        grid=(n // _BN,),
        in_specs=[
            pl.BlockSpec((_BN, _D), lambda i: (i, 0)),
            pl.BlockSpec((_D, _OUT), lambda i: (0, 0)),
            pl.BlockSpec((1, _OUT), lambda i: (0, 0)),
        ],
        out_specs=pl.BlockSpec((_BN, _OUT), lambda i: (i, 0)),
        out_shape=jax.ShapeDtypeStruct((n, _OUT), jnp.float32),
        compiler_params=pltpu.CompilerParams(
            dimension_semantics=("arbitrary",),
        ),
    )(x, w, comb)


def kernel(inputs, kernel, bias, a, c):
    comb = bias + a + c
    comb16 = jnp.pad(comb, (0, _L - _OUT))
    wt = kernel.T
    n_tc = _N - _SC_ROWS
    sc_out = _sc_call(inputs, wt, comb16, n_tc, _SC_ROWS)
    tc_out = _tc_call(inputs, kernel.astype(jnp.bfloat16),
                      comb.reshape(1, _OUT), n_tc)
    return jnp.concatenate([tc_out, sc_out], axis=0)


# speedup vs baseline: 2.9673x; 1.1597x over previous
import jax
import jax.numpy as jnp
from jax.experimental import pallas as pl
from jax.experimental.pallas import tpu as pltpu

_N, _D, _OUT = 16384, 4096, 4
_BN = 512
_DH = _D // 2


def _body(x0_ref, x1_ref, w0_ref, w1_ref, b_ref, o_ref):
    y0 = jnp.dot(x0_ref[...].astype(jnp.bfloat16), w0_ref[...],
                 preferred_element_type=jnp.float32)
    y1 = jnp.dot(x1_ref[...].astype(jnp.bfloat16), w1_ref[...],
                 preferred_element_type=jnp.float32)
    o_ref[...] = y0 + y1 + b_ref[...]


def kernel(inputs, kernel, bias, a, c):
    comb = (bias + a + c).reshape(1, _OUT)
    w = kernel.astype(jnp.bfloat16)
    w0 = w[:_DH]
    w1 = w[_DH:]
    return pl.pallas_call(
        _body,
        grid=(_N // _BN,),
        in_specs=[
            pl.BlockSpec((_BN, _DH), lambda i: (i, 0)),
            pl.BlockSpec((_BN, _DH), lambda i: (i, 1)),
            pl.BlockSpec((_DH, _OUT), lambda i: (0, 0)),
            pl.BlockSpec((_DH, _OUT), lambda i: (0, 0)),
            pl.BlockSpec((1, _OUT), lambda i: (0, 0)),
        ],
        out_specs=pl.BlockSpec((_BN, _OUT), lambda i: (i, 0)),
        out_shape=jax.ShapeDtypeStruct((_N, _OUT), jnp.float32),
        compiler_params=pltpu.CompilerParams(
            dimension_semantics=("arbitrary",),
        ),
    )(inputs, inputs, w0, w1, comb)
